# baseline (speedup 1.0000x reference)
import functools

import jax
import jax.numpy as jnp
from jax import lax
from jax.experimental import pallas as pl
from jax.experimental.pallas import tpu as pltpu
from jax.experimental.pallas import tpu_sc as plsc

D = 64
W = 128
LANES = 16
SUB = 128
NSUB = 4
WINW = 512
NWIN_REG = 1953
NWIN_PAD = 1984
WPW = 62
CH = 32
CAPC = 3520


def _p1_body(tab_hbm, tail_hbm, meta_hbm, cols_hbm, dsts_hbm, gath_hbm,
             meta_v, wbuf, tailb, colv, dstv, stage, sem):
    nc = plsc.get_sparse_core_info().num_cores
    wid = lax.axis_index("s") * nc + lax.axis_index("c")
    pltpu.sync_copy(meta_hbm.at[wid], meta_v)
    pltpu.sync_copy(tail_hbm, tailb)
    iota16 = lax.iota(jnp.int32, LANES)

    def chunks(buf, nck, cst):
        def chunk(q, _):
            off = (cst + q) * CH
            pltpu.sync_copy(cols_hbm.at[pl.ds(off, CH)], colv)
            pltpu.sync_copy(dsts_hbm.at[pl.ds(off, CH)], dstv)
            for g in range(CH // LANES):
                col16 = colv[pl.ds(g * LANES, LANES)]
                rows = g * LANES + iota16

                def dims(d, _):
                    for u in range(4):
                        dd = d * 4 + u
                        ddv = jnp.full((LANES,), dd, jnp.int32)
                        v = plsc.load_gather(buf, [ddv, col16])
                        plsc.store_scatter(stage, [rows, ddv], v)
                    return 0

                lax.fori_loop(0, D // 4, dims, 0)
            pltpu.async_copy(stage, gath_hbm.at[dstv], sem).wait()
            return 0

        lax.fori_loop(0, nck, chunk, 0)

    def win(k, _):
        row = meta_v[k, pl.ds(0, LANES)]
        c0, nck, cst, istail = row[0], row[1], row[2], row[3]

        @pl.when(istail == 0)
        def _():
            c0a = pl.multiple_of(c0, WINW)
            pltpu.sync_copy(tab_hbm.at[:, pl.ds(c0a, WINW)], wbuf)
            chunks(wbuf, nck, cst)

        @pl.when(istail == 1)
        def _():
            chunks(tailb, nck, cst)

        return 0

    lax.fori_loop(0, WPW, win, 0)


def _p2_body(gath_hbm, small_hbm, idx_hbm, out_hbm,
             idx_v, h_v, t_v, n_v, r_v, y_v, z_v, out_v, sem):
    nc = plsc.get_sparse_core_info().num_cores
    wid = lax.axis_index("s") * nc + lax.axis_index("c")
    B = 16384
    pltpu.sync_copy(idx_hbm.at[wid], idx_v)

    def sub_chunk(j, acc):
        base = wid * (NSUB * SUB) + j * SUB
        cps = [
            pltpu.async_copy(gath_hbm.at[pl.ds(base, SUB)], h_v, sem),
            pltpu.async_copy(gath_hbm.at[pl.ds(B + base, SUB)], t_v, sem),
            pltpu.async_copy(gath_hbm.at[pl.ds(2 * B + base, SUB)], n_v, sem),
            pltpu.async_copy(small_hbm.at[idx_v.at[0, j]], r_v, sem),
            pltpu.async_copy(small_hbm.at[idx_v.at[1, j]], y_v, sem),
            pltpu.async_copy(small_hbm.at[idx_v.at[2, j]], z_v, sem),
        ]
        for cp in cps:
            cp.wait()

        def row(rr, acc):
            for k in range(D // LANES):
                sl = pl.ds(k * LANES, LANES)
                c = r_v[rr, sl] + y_v[rr, sl] + z_v[rr, sl] - t_v[rr, sl]
                acc = acc + jnp.abs(n_v[rr, sl] + c) - jnp.abs(h_v[rr, sl] + c)
            return acc

        return lax.fori_loop(0, SUB, row, acc)

    acc = lax.fori_loop(0, NSUB, sub_chunk, jnp.zeros((LANES,), jnp.float32))
    out_v[...] = acc
    pltpu.sync_copy(out_v, out_hbm.at[wid])


def kernel(pos_h, pos_t, pos_r, pos_tem, ent_emb, rel_emb, year_emb,
           month_emb, day_emb, hour_emb, minutes_emb, sec_emb):
    B = pos_h.shape[0]
    n_ent = ent_emb.shape[0]
    n_rel = rel_emb.shape[0]
    info = plsc.get_sparse_core_info()
    nw = info.num_cores * info.num_subcores
    assert B == nw * NSUB * SUB and n_ent == 1000000

    neg_h = jax.random.randint(jax.random.key(1), pos_h.shape, 1, n_ent,
                               dtype=jnp.int32)

    tab = ent_emb.T
    tail_c0 = NWIN_REG * WINW
    tail_tab = ent_emb[tail_c0:, :].T

    Bq = 3 * B
    trash0 = Bq
    e = jnp.concatenate([pos_h, pos_t, neg_h]).astype(jnp.int32)
    dst_all = jnp.arange(Bq, dtype=jnp.int32)
    win = jnp.minimum(e >> 9, NWIN_REG)
    c0_of_win = jnp.minimum(jnp.arange(NWIN_PAD, dtype=jnp.int32) * WINW,
                            tail_c0)
    local = e - c0_of_win[win]
    order = jnp.argsort(win, stable=True)
    wsorted = win[order]
    cnt = jnp.zeros((NWIN_PAD,), jnp.int32).at[win].add(1)
    nck = (cnt + CH - 1) // CH
    cstart = jnp.cumsum(nck) - nck
    wstart = jnp.cumsum(cnt) - cnt
    pos_in = jnp.arange(Bq, dtype=jnp.int32) - wstart[wsorted]
    slot = (cstart[wsorted] + pos_in // CH) * CH + pos_in % CH
    wchunk = jnp.repeat(jnp.arange(NWIN_PAD, dtype=jnp.int32), nck,
                        total_repeat_length=CAPC)
    dst_dflt = jnp.broadcast_to((trash0 + wchunk // WPW)[:, None],
                                (CAPC, CH)).reshape(-1)
    cols_arr = jnp.zeros((CAPC * CH,), jnp.int32).at[slot].set(local[order])
    dsts_arr = dst_dflt.at[slot].set(dst_all[order])
    istail = (jnp.arange(NWIN_PAD, dtype=jnp.int32) >= NWIN_REG)
    meta = jnp.stack([c0_of_win, nck.astype(jnp.int32),
                      cstart.astype(jnp.int32), istail.astype(jnp.int32)],
                     axis=1)
    meta = jnp.pad(meta, ((0, 0), (0, 12))).reshape(nw, WPW, 16)

    ymd = (year_emb[:13, None, None, :] + month_emb[None, :13, None, :]
           + day_emb[None, None, :13, :]).reshape(13 * 13 * 13, D)
    hms = (hour_emb[:13, None, None, :] + minutes_emb[None, :13, None, :]
           + sec_emb[None, None, :13, :]).reshape(13 * 13 * 13, D)
    small = jnp.concatenate([rel_emb, ymd, hms], axis=0)
    small = jnp.pad(small, ((0, 0), (0, W - D)))

    ymd_idx = n_rel + (pos_tem[:, 0] * 169 + pos_tem[:, 1] * 13 + pos_tem[:, 2])
    hms_idx = (n_rel + 2197
               + (pos_tem[:, 3] * 169 + pos_tem[:, 4] * 13 + pos_tem[:, 5]))
    idx3 = (jnp.stack([pos_r, ymd_idx, hms_idx]).astype(jnp.int32)
            .reshape(3, nw, NSUB, SUB).transpose(1, 0, 2, 3))

    mesh = plsc.VectorSubcoreMesh(core_axis_name="c", subcore_axis_name="s")
    cparams = pltpu.CompilerParams(use_tc_tiling_on_sc=True,
                                   needs_layout_passes=False)

    p1 = functools.partial(
        pl.kernel,
        mesh=mesh,
        compiler_params=cparams,
        out_type=jax.ShapeDtypeStruct((Bq + nw, W), jnp.float32),
        scratch_types=[
            pltpu.VMEM((WPW, 16), jnp.int32),
            pltpu.VMEM((D, WINW), jnp.float32),
            pltpu.VMEM((D, D), jnp.float32),
            pltpu.VMEM((CH,), jnp.int32),
            pltpu.VMEM((CH,), jnp.int32),
            pltpu.VMEM((CH, W), jnp.float32),
            pltpu.SemaphoreType.DMA,
        ],
    )(_p1_body)
    gath = p1(tab, tail_tab, meta, cols_arr, dsts_arr)

    p2 = functools.partial(
        pl.kernel,
        mesh=mesh,
        compiler_params=cparams,
        out_type=jax.ShapeDtypeStruct((nw, LANES), jnp.float32),
        scratch_types=[
            pltpu.VMEM((3, NSUB, SUB), jnp.int32),
            pltpu.VMEM((SUB, W), jnp.float32),
            pltpu.VMEM((SUB, W), jnp.float32),
            pltpu.VMEM((SUB, W), jnp.float32),
            pltpu.VMEM((SUB, W), jnp.float32),
            pltpu.VMEM((SUB, W), jnp.float32),
            pltpu.VMEM((SUB, W), jnp.float32),
            pltpu.VMEM((LANES,), jnp.float32),
            pltpu.SemaphoreType.DMA,
        ],
    )(_p2_body)
    partials = p2(gath, small, idx3)
    return 1.0 + jnp.sum(partials) / B


# speedup vs baseline: 2.6022x; 2.6022x over previous
import functools

import jax
import jax.numpy as jnp
from jax import lax
from jax.experimental import pallas as pl
from jax.experimental.pallas import tpu as pltpu
from jax.experimental.pallas import tpu_sc as plsc

D = 64
W = 128
LANES = 16
SUB = 128
NSUB = 4
WINW = 512
NWIN_REG = 1953
NWIN_PAD = 1984
WPW = 62
CH = 32
CAPC = 3520


def _p1_body(tab_hbm, tail_hbm, meta_hbm, cols_hbm, dsts_hbm, gath_hbm,
             meta_v, wbuf, tailb, colv, dstv, dstv2, stage, sem):
    nc = plsc.get_sparse_core_info().num_cores
    wid = lax.axis_index("s") * nc + lax.axis_index("c")
    pltpu.sync_copy(meta_hbm.at[wid], meta_v)
    pltpu.sync_copy(tail_hbm, tailb)
    iota16 = lax.iota(jnp.int32, LANES)

    trash = jnp.int32(3 * 16384) + wid

    def chunks(buf, q0, nq, ws, we, clamp):
        def chunk(q, _):
            off = (q0 + q) * CH
            pltpu.sync_copy(cols_hbm.at[pl.ds(off * 1, CH)], colv)
            pltpu.sync_copy(dsts_hbm.at[pl.ds(off * 1, CH)], dstv)
            for g in range(CH // LANES):
                col16 = colv[pl.ds(g * LANES, LANES)]
                if clamp:
                    col16 = jnp.minimum(col16, D - 1)
                gid = off + g * LANES + iota16
                m = (gid >= ws) & (gid < we)
                dsel = jnp.where(m, dstv[pl.ds(g * LANES, LANES)], trash)
                dstv2[pl.ds(g * LANES, LANES)] = dsel
                rows = g * LANES + iota16

                def dims(d, _):
                    for u in range(4):
                        dd = d * 4 + u
                        ddv = jnp.full((LANES,), dd, jnp.int32)
                        v = plsc.load_gather(buf, [ddv, col16])
                        plsc.store_scatter(stage, [rows, ddv], v)
                    return 0

                lax.fori_loop(0, D // 4, dims, 0)
            pltpu.async_copy(stage, gath_hbm.at[dstv2], sem).wait()
            return 0

        lax.fori_loop(0, nq, chunk, 0)

    def win(k, _):
        row = meta_v[k, pl.ds(0, LANES)]
        c0, q0, nq = row[0], row[1], row[2]
        ws, we, istail = row[3], row[4], row[5]

        @pl.when((istail == 0) & (nq > 0))
        def _():
            c0a = pl.multiple_of(c0, WINW)
            pltpu.sync_copy(tab_hbm.at[:, pl.ds(c0a, WINW)], wbuf)
            chunks(wbuf, q0, nq, ws, we, False)

        @pl.when(istail == 1)
        def _():
            chunks(tailb, q0, nq, ws, we, True)

        return 0

    lax.fori_loop(0, WPW, win, 0)


def _p2_body(gath_hbm, small_hbm, idx_hbm, out_hbm,
             idx_v, h_v, t_v, n_v, r_v, y_v, z_v, out_v, sem):
    nc = plsc.get_sparse_core_info().num_cores
    wid = lax.axis_index("s") * nc + lax.axis_index("c")
    B = 16384
    pltpu.sync_copy(idx_hbm.at[wid], idx_v)

    def sub_chunk(j, acc):
        base = wid * (NSUB * SUB) + j * SUB
        cps = [
            pltpu.async_copy(gath_hbm.at[pl.ds(base, SUB)], h_v, sem),
            pltpu.async_copy(gath_hbm.at[pl.ds(B + base, SUB)], t_v, sem),
            pltpu.async_copy(gath_hbm.at[pl.ds(2 * B + base, SUB)], n_v, sem),
            pltpu.async_copy(small_hbm.at[idx_v.at[0, j]], r_v, sem),
            pltpu.async_copy(small_hbm.at[idx_v.at[1, j]], y_v, sem),
            pltpu.async_copy(small_hbm.at[idx_v.at[2, j]], z_v, sem),
        ]
        for cp in cps:
            cp.wait()

        def row(rr, acc):
            for k in range(D // LANES):
                sl = pl.ds(k * LANES, LANES)
                c = r_v[rr, sl] + y_v[rr, sl] + z_v[rr, sl] - t_v[rr, sl]
                acc = acc + jnp.abs(n_v[rr, sl] + c) - jnp.abs(h_v[rr, sl] + c)
            return acc

        return lax.fori_loop(0, SUB, row, acc)

    acc = lax.fori_loop(0, NSUB, sub_chunk, jnp.zeros((LANES,), jnp.float32))
    out_v[...] = acc
    pltpu.sync_copy(out_v, out_hbm.at[wid])


def kernel(pos_h, pos_t, pos_r, pos_tem, ent_emb, rel_emb, year_emb,
           month_emb, day_emb, hour_emb, minutes_emb, sec_emb):
    B = pos_h.shape[0]
    n_ent = ent_emb.shape[0]
    n_rel = rel_emb.shape[0]
    info = plsc.get_sparse_core_info()
    nw = info.num_cores * info.num_subcores
    assert B == nw * NSUB * SUB and n_ent == 1000000

    neg_h = jax.random.randint(jax.random.key(1), pos_h.shape, 1, n_ent,
                               dtype=jnp.int32)

    tab = ent_emb.T
    tail_c0 = NWIN_REG * WINW
    tail_tab = ent_emb[tail_c0:, :].T

    Bq = 3 * B
    e = jnp.concatenate([pos_h, pos_t, neg_h]).astype(jnp.int32)
    dst_all = jnp.arange(Bq, dtype=jnp.int32)
    es, dsts_arr = jax.lax.sort([e, dst_all], num_keys=1)
    wsorted = jnp.minimum(es >> 9, NWIN_REG)
    cols_arr = es - jnp.minimum(wsorted * WINW, NWIN_REG * WINW)
    warr = jnp.arange(NWIN_PAD, dtype=jnp.int32)
    wstart = jnp.searchsorted(wsorted, warr, side="left").astype(jnp.int32)
    wend = jnp.searchsorted(wsorted, warr, side="right").astype(jnp.int32)
    cnt = wend - wstart
    q0 = wstart // CH
    nq = jnp.where(cnt > 0, (wend + CH - 1) // CH - q0, 0)
    c0_of_win = jnp.minimum(warr * WINW, NWIN_REG * WINW)
    istail = (warr >= NWIN_REG).astype(jnp.int32)
    meta = jnp.stack([c0_of_win, q0.astype(jnp.int32), nq.astype(jnp.int32),
                      wstart, wend, istail], axis=1)
    meta = jnp.pad(meta, ((0, 0), (0, 10))).reshape(nw, WPW, 16)

    ymd = (year_emb[:13, None, None, :] + month_emb[None, :13, None, :]
           + day_emb[None, None, :13, :]).reshape(13 * 13 * 13, D)
    hms = (hour_emb[:13, None, None, :] + minutes_emb[None, :13, None, :]
           + sec_emb[None, None, :13, :]).reshape(13 * 13 * 13, D)
    small = jnp.concatenate([rel_emb, ymd, hms], axis=0)
    small = jnp.pad(small, ((0, 0), (0, W - D)))

    ymd_idx = n_rel + (pos_tem[:, 0] * 169 + pos_tem[:, 1] * 13 + pos_tem[:, 2])
    hms_idx = (n_rel + 2197
               + (pos_tem[:, 3] * 169 + pos_tem[:, 4] * 13 + pos_tem[:, 5]))
    idx3 = (jnp.stack([pos_r, ymd_idx, hms_idx]).astype(jnp.int32)
            .reshape(3, nw, NSUB, SUB).transpose(1, 0, 2, 3))

    mesh = plsc.VectorSubcoreMesh(core_axis_name="c", subcore_axis_name="s")
    cparams = pltpu.CompilerParams(use_tc_tiling_on_sc=True,
                                   needs_layout_passes=False)

    p1 = functools.partial(
        pl.kernel,
        mesh=mesh,
        compiler_params=cparams,
        out_type=jax.ShapeDtypeStruct((Bq + nw, W), jnp.float32),
        scratch_types=[
            pltpu.VMEM((WPW, 16), jnp.int32),
            pltpu.VMEM((D, WINW), jnp.float32),
            pltpu.VMEM((D, D), jnp.float32),
            pltpu.VMEM((CH,), jnp.int32),
            pltpu.VMEM((CH,), jnp.int32),
            pltpu.VMEM((CH,), jnp.int32),
            pltpu.VMEM((CH, W), jnp.float32),
            pltpu.SemaphoreType.DMA,
        ],
    )(_p1_body)
    gath = p1(tab, tail_tab, meta, cols_arr, dsts_arr)

    p2 = functools.partial(
        pl.kernel,
        mesh=mesh,
        compiler_params=cparams,
        out_type=jax.ShapeDtypeStruct((nw, LANES), jnp.float32),
        scratch_types=[
            pltpu.VMEM((3, NSUB, SUB), jnp.int32),
            pltpu.VMEM((SUB, W), jnp.float32),
            pltpu.VMEM((SUB, W), jnp.float32),
            pltpu.VMEM((SUB, W), jnp.float32),
            pltpu.VMEM((SUB, W), jnp.float32),
            pltpu.VMEM((SUB, W), jnp.float32),
            pltpu.VMEM((SUB, W), jnp.float32),
            pltpu.VMEM((LANES,), jnp.float32),
            pltpu.SemaphoreType.DMA,
        ],
    )(_p2_body)
    partials = p2(gath, small, idx3)
    return 1.0 + jnp.sum(partials) / B


# speedup vs baseline: 2.7360x; 1.0514x over previous
import functools

import jax
import jax.numpy as jnp
from jax import lax
from jax.experimental import pallas as pl
from jax.experimental.pallas import tpu as pltpu
from jax.experimental.pallas import tpu_sc as plsc

D = 64
W = 128
LANES = 16
SUB = 128
NSUB = 4
WINW = 512
NWIN_REG = 1953
NWIN_PAD = 1984
WPW = 62
CH = 32
B = 16384
Bq = 3 * B


def _p1_body(tab_hbm, tail_hbm, meta_hbm, cd_hbm, gath_hbm,
             meta_v, wbA, wbB, tailb, cdA, cdB, cdR, dstA, dstB, dstR,
             stgA, stgB, stgR,
             semSA, semSB, semCA, semCB, semWA, semWB, semR):
    nc = plsc.get_sparse_core_info().num_cores
    wid = lax.axis_index("s") * nc + lax.axis_index("c")
    pltpu.sync_copy(meta_hbm.at[wid], meta_v)
    pltpu.sync_copy(tail_hbm, tailb)
    iota16 = lax.iota(jnp.int32, LANES)
    trash = jnp.int32(Bq) + wid

    def getrow(k):
        row = meta_v[k, pl.ds(0, LANES)]
        return row[0], row[1], row[2], row[3], row[4], row[5]

    def fire(k, wb, cdv, semS, semC):
        c0, q0, nq, ws, we, it = getrow(k)

        @pl.when((it == 0) & (nq > 0))
        def _():
            c0a = pl.multiple_of(c0, WINW)
            pltpu.async_copy(tab_hbm.at[:, pl.ds(c0a, WINW)], wb, semS)

        @pl.when(nq > 0)
        def _():
            pltpu.async_copy(cd_hbm.at[pl.ds(q0 * 2 * CH, 4 * CH)], cdv, semC)

    def extract(buf, col16, stg, rowbase, clamp):
        if clamp:
            col16 = jnp.minimum(col16, D - 1)
        rows = rowbase + iota16

        def dims(d, _):
            for u in range(4):
                dd = d * 4 + u
                ddv = jnp.full((LANES,), dd, jnp.int32)
                v = plsc.load_gather(buf, [ddv, col16])
                plsc.store_scatter(stg, [rows, ddv], v)
            return 0

        lax.fori_loop(0, D // 4, dims, 0)

    def window(parity, k, prev_k, fire_k):
        wb, cdv, dstv, stg = ((wbA, cdA, dstA, stgA) if parity == 0
                              else (wbB, cdB, dstB, stgB))
        semS, semC, semW = ((semSA, semCA, semWA) if parity == 0
                            else (semSB, semCB, semWB))
        c0, q0, nq, ws, we, it = getrow(k)

        _, _, pnq, _, _, _ = getrow(jnp.maximum(prev_k, 0))

        @pl.when((prev_k >= 0) & (pnq > 0))
        def _():
            pltpu.make_async_copy(stg, gath_hbm.at[pl.ds(0, 2 * CH)],
                                  semW).wait()

        @pl.when((it == 0) & (nq > 0))
        def _():
            pltpu.make_async_copy(tab_hbm.at[:, pl.ds(0, WINW)], wb,
                                  semS).wait()

        @pl.when(nq > 0)
        def _():
            pltpu.make_async_copy(cd_hbm.at[pl.ds(0, 4 * CH)], cdv,
                                  semC).wait()

        fire(fire_k, wbB if parity == 0 else wbA,
             cdB if parity == 0 else cdA,
             semSB if parity == 0 else semSA,
             semCB if parity == 0 else semCA)

        def body(buf, clamp):
            for q in range(2):
                for g in range(CH // LANES):
                    sl = pl.ds(q * CH + g * LANES, LANES)
                    col16 = cdv[pl.ds(q * 2 * CH + g * LANES, LANES)]
                    dvec = cdv[pl.ds(q * 2 * CH + CH + g * LANES, LANES)]
                    gid = (q0 + q) * CH + g * LANES + iota16
                    m = (gid >= ws) & (gid < we)
                    dstv[sl] = jnp.where(m, dvec, trash)
                    extract(buf, col16, stg, q * CH + g * LANES, clamp)
            pltpu.async_copy(stg, gath_hbm.at[dstv], semW)

            @pl.when(nq > 2)
            def _():
                def rchunk(q, _):
                    pltpu.sync_copy(
                        cd_hbm.at[pl.ds((q0 + q) * 2 * CH, 2 * CH)], cdR)
                    for g in range(CH // LANES):
                        sl = pl.ds(g * LANES, LANES)
                        col16 = cdR[pl.ds(g * LANES, LANES)]
                        dvec = cdR[pl.ds(CH + g * LANES, LANES)]
                        gid = (q0 + q) * CH + g * LANES + iota16
                        m = (gid >= ws) & (gid < we)
                        dstR[sl] = jnp.where(m, dvec, trash)
                        extract(buf, col16, stgR, g * LANES, clamp)
                    pltpu.async_copy(stgR, gath_hbm.at[dstR], semR).wait()
                    return 0

                lax.fori_loop(2, nq, rchunk, 0)

        @pl.when((it == 0) & (nq > 0))
        def _():
            body(wb, False)

        @pl.when((it == 1) & (nq > 0))
        def _():
            body(tailb, True)

    fire(0, wbA, cdA, semSA, semCA)

    def pair(p, _):
        k0 = 2 * p
        k1 = 2 * p + 1
        window(0, k0, k0 - 2, k1)
        window(1, k1, k1 - 2, k1 + 1)
        return 0

    lax.fori_loop(0, WPW // 2, pair, 0)

    for parity, klast in ((0, WPW - 2), (1, WPW - 1)):
        _, _, lnq, _, _, _ = getrow(klast)
        stg = stgA if parity == 0 else stgB
        semW = semWA if parity == 0 else semWB

        @pl.when(lnq > 0)
        def _():
            pltpu.make_async_copy(stg, gath_hbm.at[pl.ds(0, 2 * CH)],
                                  semW).wait()


def _p2_body(gath_hbm, small_hbm, idx_hbm, out_hbm,
             idx_v, h_v, t_v, n_v, r_v, y_v, z_v, out_v, sem):
    nc = plsc.get_sparse_core_info().num_cores
    wid = lax.axis_index("s") * nc + lax.axis_index("c")
    pltpu.sync_copy(idx_hbm.at[wid], idx_v)

    def sub_chunk(j, acc):
        base = wid * (NSUB * SUB) + j * SUB
        cps = [
            pltpu.async_copy(gath_hbm.at[pl.ds(base, SUB)], h_v, sem),
            pltpu.async_copy(gath_hbm.at[pl.ds(B + base, SUB)], t_v, sem),
            pltpu.async_copy(gath_hbm.at[pl.ds(2 * B + base, SUB)], n_v, sem),
            pltpu.async_copy(small_hbm.at[idx_v.at[0, j]], r_v, sem),
            pltpu.async_copy(small_hbm.at[idx_v.at[1, j]], y_v, sem),
            pltpu.async_copy(small_hbm.at[idx_v.at[2, j]], z_v, sem),
        ]
        for cp in cps:
            cp.wait()

        def row(rr, acc):
            for k in range(D // LANES):
                sl = pl.ds(k * LANES, LANES)
                c = r_v[rr, sl] + y_v[rr, sl] + z_v[rr, sl] - t_v[rr, sl]
                acc = acc + jnp.abs(n_v[rr, sl] + c) - jnp.abs(h_v[rr, sl] + c)
            return acc

        return lax.fori_loop(0, SUB, row, acc)

    acc = lax.fori_loop(0, NSUB, sub_chunk, jnp.zeros((LANES,), jnp.float32))
    out_v[...] = acc
    pltpu.sync_copy(out_v, out_hbm.at[wid])


def kernel(pos_h, pos_t, pos_r, pos_tem, ent_emb, rel_emb, year_emb,
           month_emb, day_emb, hour_emb, minutes_emb, sec_emb):
    n_ent = ent_emb.shape[0]
    n_rel = rel_emb.shape[0]
    info = plsc.get_sparse_core_info()
    nw = info.num_cores * info.num_subcores
    assert pos_h.shape[0] == B == nw * NSUB * SUB and n_ent == 1000000

    neg_h = jax.random.randint(jax.random.key(1), pos_h.shape, 1, n_ent,
                               dtype=jnp.int32)

    tab = ent_emb.T
    tail_c0 = NWIN_REG * WINW
    tail_tab = ent_emb[tail_c0:, :].T

    e = jnp.concatenate([pos_h, pos_t, neg_h]).astype(jnp.int32)
    dst_all = jnp.arange(Bq, dtype=jnp.int32)
    es, dsts_arr = jax.lax.sort([e, dst_all], num_keys=1)
    wsorted = jnp.minimum(es >> 9, NWIN_REG)
    cols_arr = es - jnp.minimum(wsorted * WINW, tail_c0)
    cd = jnp.stack([cols_arr.reshape(-1, CH),
                    dsts_arr.reshape(-1, CH)], axis=1).reshape(-1)
    cd = jnp.pad(cd, (0, 4 * CH))
    warr = jnp.arange(NWIN_PAD, dtype=jnp.int32)
    wstart = jnp.searchsorted(wsorted, warr, side="left",
                              method="sort").astype(jnp.int32)
    wend = jnp.concatenate([wstart[1:], jnp.full((1,), Bq, jnp.int32)])
    cnt = wend - wstart
    q0 = wstart // CH
    nq = jnp.where(cnt > 0, (wend + CH - 1) // CH - q0, 0)
    c0_of_win = jnp.minimum(warr * WINW, tail_c0)
    istail = (warr >= NWIN_REG).astype(jnp.int32)
    meta = jnp.stack([c0_of_win, q0, nq.astype(jnp.int32),
                      wstart, wend, istail], axis=1)
    meta = meta.reshape(nw, WPW, 6)
    meta = jnp.pad(meta, ((0, 0), (0, 1), (0, 10)))
    meta = meta.at[:, WPW, :].set(0)

    ymd = (year_emb[:13, None, None, :] + month_emb[None, :13, None, :]
           + day_emb[None, None, :13, :]).reshape(13 * 13 * 13, D)
    hms = (hour_emb[:13, None, None, :] + minutes_emb[None, :13, None, :]
           + sec_emb[None, None, :13, :]).reshape(13 * 13 * 13, D)
    small = jnp.concatenate([rel_emb, ymd, hms], axis=0)
    small = jnp.pad(small, ((0, 0), (0, W - D)))

    ymd_idx = n_rel + (pos_tem[:, 0] * 169 + pos_tem[:, 1] * 13 + pos_tem[:, 2])
    hms_idx = (n_rel + 2197
               + (pos_tem[:, 3] * 169 + pos_tem[:, 4] * 13 + pos_tem[:, 5]))
    idx3 = (jnp.stack([pos_r, ymd_idx, hms_idx]).astype(jnp.int32)
            .reshape(3, nw, NSUB, SUB).transpose(1, 0, 2, 3))

    mesh = plsc.VectorSubcoreMesh(core_axis_name="c", subcore_axis_name="s")
    cparams = pltpu.CompilerParams(use_tc_tiling_on_sc=True,
                                   needs_layout_passes=False)

    p1 = functools.partial(
        pl.kernel,
        mesh=mesh,
        compiler_params=cparams,
        out_type=jax.ShapeDtypeStruct((Bq + nw, W), jnp.float32),
        scratch_types=[
            pltpu.VMEM((WPW + 1, 16), jnp.int32),
            pltpu.VMEM((D, WINW), jnp.float32),
            pltpu.VMEM((D, WINW), jnp.float32),
            pltpu.VMEM((D, D), jnp.float32),
            pltpu.VMEM((4 * CH,), jnp.int32),
            pltpu.VMEM((4 * CH,), jnp.int32),
            pltpu.VMEM((2 * CH,), jnp.int32),
            pltpu.VMEM((2 * CH,), jnp.int32),
            pltpu.VMEM((2 * CH,), jnp.int32),
            pltpu.VMEM((CH,), jnp.int32),
            pltpu.VMEM((2 * CH, W), jnp.float32),
            pltpu.VMEM((2 * CH, W), jnp.float32),
            pltpu.VMEM((CH, W), jnp.float32),
            pltpu.SemaphoreType.DMA,
            pltpu.SemaphoreType.DMA,
            pltpu.SemaphoreType.DMA,
            pltpu.SemaphoreType.DMA,
            pltpu.SemaphoreType.DMA,
            pltpu.SemaphoreType.DMA,
            pltpu.SemaphoreType.DMA,
        ],
    )(_p1_body)
    gath = p1(tab, tail_tab, meta, cd)

    p2 = functools.partial(
        pl.kernel,
        mesh=mesh,
        compiler_params=cparams,
        out_type=jax.ShapeDtypeStruct((nw, LANES), jnp.float32),
        scratch_types=[
            pltpu.VMEM((3, NSUB, SUB), jnp.int32),
            pltpu.VMEM((SUB, W), jnp.float32),
            pltpu.VMEM((SUB, W), jnp.float32),
            pltpu.VMEM((SUB, W), jnp.float32),
            pltpu.VMEM((SUB, W), jnp.float32),
            pltpu.VMEM((SUB, W), jnp.float32),
            pltpu.VMEM((SUB, W), jnp.float32),
            pltpu.VMEM((LANES,), jnp.float32),
            pltpu.SemaphoreType.DMA,
        ],
    )(_p2_body)
    partials = p2(gath, small, idx3)
    return 1.0 + jnp.sum(partials) / B


# speedup vs baseline: 3.7291x; 1.3630x over previous
import functools

import jax
import jax.numpy as jnp
from jax import lax
from jax.experimental import pallas as pl
from jax.experimental.pallas import tpu as pltpu
from jax.experimental.pallas import tpu_sc as plsc

D = 64
W = 128
LANES = 16
SUB = 128
NSUB = 4
WINW = 512
NWIN_REG = 1953
NWIN_PAD = 1984
WPW = 62
CH = 32
B = 16384
Bq = 3 * B


def _p1_body(tab_hbm, tail_hbm, meta_hbm, cd_hbm, gath_hbm,
             meta_v, wbA, wbB, tailb, cdA, cdB, cdR, dstA, dstB, dstR,
             stgA, stgB, stgR,
             semSA, semSB, semCA, semCB, semWA, semWB, semR):
    nc = plsc.get_sparse_core_info().num_cores
    wid = lax.axis_index("s") * nc + lax.axis_index("c")
    pltpu.sync_copy(meta_hbm.at[wid], meta_v)
    pltpu.sync_copy(tail_hbm, tailb)
    iota16 = lax.iota(jnp.int32, LANES)
    trash = jnp.int32(Bq) + wid

    def getrow(k):
        row = meta_v[k, pl.ds(0, LANES)]
        return row[0], row[1], row[2], row[3], row[4], row[5]

    def fire(k, wb, cdv, semS, semC):
        c0, q0, nq, ws, we, it = getrow(k)

        @pl.when((it == 0) & (nq > 0))
        def _():
            c0a = pl.multiple_of(c0, WINW)
            pltpu.async_copy(tab_hbm.at[:, pl.ds(c0a, WINW)], wb, semS)

        @pl.when(nq > 0)
        def _():
            pltpu.async_copy(cd_hbm.at[pl.ds(q0 * 2 * CH, 4 * CH)], cdv, semC)

    def extract(buf, col16, stg, rowbase, clamp):
        if clamp:
            col16 = jnp.minimum(col16, D - 1)
        rows = rowbase + iota16

        def dims(d, _):
            for u in range(4):
                dd = d * 4 + u
                ddv = jnp.full((LANES,), dd, jnp.int32)
                v = plsc.load_gather(buf, [ddv, col16])
                plsc.store_scatter(stg, [rows, ddv], v)
            return 0

        lax.fori_loop(0, D // 4, dims, 0)

    def window(parity, k, prev_k, fire_k):
        wb, cdv, dstv, stg = ((wbA, cdA, dstA, stgA) if parity == 0
                              else (wbB, cdB, dstB, stgB))
        semS, semC, semW = ((semSA, semCA, semWA) if parity == 0
                            else (semSB, semCB, semWB))
        c0, q0, nq, ws, we, it = getrow(k)

        _, _, pnq, _, _, _ = getrow(jnp.maximum(prev_k, 0))

        @pl.when((prev_k >= 0) & (pnq > 0))
        def _():
            pltpu.make_async_copy(stg, gath_hbm.at[pl.ds(0, 2 * CH)],
                                  semW).wait()

        @pl.when((it == 0) & (nq > 0))
        def _():
            pltpu.make_async_copy(tab_hbm.at[:, pl.ds(0, WINW)], wb,
                                  semS).wait()

        @pl.when(nq > 0)
        def _():
            pltpu.make_async_copy(cd_hbm.at[pl.ds(0, 4 * CH)], cdv,
                                  semC).wait()

        fire(fire_k, wbB if parity == 0 else wbA,
             cdB if parity == 0 else cdA,
             semSB if parity == 0 else semSA,
             semCB if parity == 0 else semCA)

        def body(buf, clamp):
            for q in range(2):
                for g in range(CH // LANES):
                    sl = pl.ds(q * CH + g * LANES, LANES)
                    col16 = cdv[pl.ds(q * 2 * CH + g * LANES, LANES)]
                    dvec = cdv[pl.ds(q * 2 * CH + CH + g * LANES, LANES)]
                    gid = (q0 + q) * CH + g * LANES + iota16
                    m = (gid >= ws) & (gid < we)
                    dstv[sl] = jnp.where(m, dvec, trash)
                    extract(buf, col16, stg, q * CH + g * LANES, clamp)
            pltpu.async_copy(stg, gath_hbm.at[dstv], semW)

            @pl.when(nq > 2)
            def _():
                def rchunk(q, _):
                    pltpu.sync_copy(
                        cd_hbm.at[pl.ds((q0 + q) * 2 * CH, 2 * CH)], cdR)
                    for g in range(CH // LANES):
                        sl = pl.ds(g * LANES, LANES)
                        col16 = cdR[pl.ds(g * LANES, LANES)]
                        dvec = cdR[pl.ds(CH + g * LANES, LANES)]
                        gid = (q0 + q) * CH + g * LANES + iota16
                        m = (gid >= ws) & (gid < we)
                        dstR[sl] = jnp.where(m, dvec, trash)
                        extract(buf, col16, stgR, g * LANES, clamp)
                    pltpu.async_copy(stgR, gath_hbm.at[dstR], semR).wait()
                    return 0

                lax.fori_loop(2, nq, rchunk, 0)

        @pl.when((it == 0) & (nq > 0))
        def _():
            body(wb, False)

        @pl.when((it == 1) & (nq > 0))
        def _():
            body(tailb, True)

    fire(0, wbA, cdA, semSA, semCA)

    def pair(p, _):
        k0 = 2 * p
        k1 = 2 * p + 1
        window(0, k0, k0 - 2, k1)
        window(1, k1, k1 - 2, k1 + 1)
        return 0

    lax.fori_loop(0, WPW // 2, pair, 0)

    for parity, klast in ((0, WPW - 2), (1, WPW - 1)):
        _, _, lnq, _, _, _ = getrow(klast)
        stg = stgA if parity == 0 else stgB
        semW = semWA if parity == 0 else semWB

        @pl.when(lnq > 0)
        def _():
            pltpu.make_async_copy(stg, gath_hbm.at[pl.ds(0, 2 * CH)],
                                  semW).wait()


def _p2_body(gath_hbm, small_hbm, idx_hbm, out_hbm,
             idx_v, h_v, t_v, n_v, r_v, y_v, z_v, out_v, sem):
    nc = plsc.get_sparse_core_info().num_cores
    wid = lax.axis_index("s") * nc + lax.axis_index("c")
    pltpu.sync_copy(idx_hbm.at[wid], idx_v)

    def sub_chunk(j, acc):
        base = wid * (NSUB * SUB) + j * SUB
        cps = [
            pltpu.async_copy(gath_hbm.at[pl.ds(base, SUB)], h_v, sem),
            pltpu.async_copy(gath_hbm.at[pl.ds(B + base, SUB)], t_v, sem),
            pltpu.async_copy(gath_hbm.at[pl.ds(2 * B + base, SUB)], n_v, sem),
            pltpu.async_copy(small_hbm.at[idx_v.at[0, j]], r_v, sem),
            pltpu.async_copy(small_hbm.at[idx_v.at[1, j]], y_v, sem),
            pltpu.async_copy(small_hbm.at[idx_v.at[2, j]], z_v, sem),
        ]
        for cp in cps:
            cp.wait()

        def row(rr, acc):
            for k in range(D // LANES):
                sl = pl.ds(k * LANES, LANES)
                c = r_v[rr, sl] + y_v[rr, sl] + z_v[rr, sl] - t_v[rr, sl]
                acc = acc + jnp.abs(n_v[rr, sl] + c) - jnp.abs(h_v[rr, sl] + c)
            return acc

        return lax.fori_loop(0, SUB, row, acc)

    acc = lax.fori_loop(0, NSUB, sub_chunk, jnp.zeros((LANES,), jnp.float32))
    out_v[...] = acc
    pltpu.sync_copy(out_v, out_hbm.at[wid])


def kernel(pos_h, pos_t, pos_r, pos_tem, ent_emb, rel_emb, year_emb,
           month_emb, day_emb, hour_emb, minutes_emb, sec_emb):
    n_ent = ent_emb.shape[0]
    n_rel = rel_emb.shape[0]
    info = plsc.get_sparse_core_info()
    nw = info.num_cores * info.num_subcores
    assert pos_h.shape[0] == B == nw * NSUB * SUB and n_ent == 1000000

    neg_h = jax.random.randint(jax.random.key(1), pos_h.shape, 1, n_ent,
                               dtype=jnp.int32)

    tab = ent_emb.T
    tail_c0 = NWIN_REG * WINW
    tail_tab = ent_emb[tail_c0:, :].T

    e = jnp.concatenate([pos_h, pos_t, neg_h]).astype(jnp.int32)
    dst_all = jnp.arange(Bq, dtype=jnp.int32)
    es, dsts_arr = jax.lax.sort([e, dst_all], num_keys=1)
    wsorted = jnp.minimum(es >> 9, NWIN_REG)
    cols_arr = es - jnp.minimum(wsorted * WINW, tail_c0)
    cd = jnp.stack([cols_arr.reshape(-1, CH),
                    dsts_arr.reshape(-1, CH)], axis=1).reshape(-1)
    cd = jnp.pad(cd, (0, 4 * CH))
    warr = jnp.arange(NWIN_PAD, dtype=jnp.int32)
    wstart = jnp.searchsorted(wsorted, warr, side="left",
                              method="scan_unrolled").astype(jnp.int32)
    wend = jnp.concatenate([wstart[1:], jnp.full((1,), Bq, jnp.int32)])
    cnt = wend - wstart
    q0 = wstart // CH
    nq = jnp.where(cnt > 0, (wend + CH - 1) // CH - q0, 0)
    c0_of_win = jnp.minimum(warr * WINW, tail_c0)
    istail = (warr >= NWIN_REG).astype(jnp.int32)
    meta = jnp.stack([c0_of_win, q0, nq.astype(jnp.int32),
                      wstart, wend, istail], axis=1)
    meta = meta.reshape(nw, WPW, 6)
    meta = jnp.pad(meta, ((0, 0), (0, 1), (0, 10)))
    meta = meta.at[:, WPW, :].set(0)

    ymd = (year_emb[:13, None, None, :] + month_emb[None, :13, None, :]
           + day_emb[None, None, :13, :]).reshape(13 * 13 * 13, D)
    hms = (hour_emb[:13, None, None, :] + minutes_emb[None, :13, None, :]
           + sec_emb[None, None, :13, :]).reshape(13 * 13 * 13, D)
    small = jnp.concatenate([rel_emb, ymd, hms], axis=0)
    small = jnp.pad(small, ((0, 0), (0, W - D)))

    ymd_idx = n_rel + (pos_tem[:, 0] * 169 + pos_tem[:, 1] * 13 + pos_tem[:, 2])
    hms_idx = (n_rel + 2197
               + (pos_tem[:, 3] * 169 + pos_tem[:, 4] * 13 + pos_tem[:, 5]))
    idx3 = (jnp.stack([pos_r, ymd_idx, hms_idx]).astype(jnp.int32)
            .reshape(3, nw, NSUB, SUB).transpose(1, 0, 2, 3))

    mesh = plsc.VectorSubcoreMesh(core_axis_name="c", subcore_axis_name="s")
    cparams = pltpu.CompilerParams(use_tc_tiling_on_sc=True,
                                   needs_layout_passes=False)

    p1 = functools.partial(
        pl.kernel,
        mesh=mesh,
        compiler_params=cparams,
        out_type=jax.ShapeDtypeStruct((Bq + nw, W), jnp.float32),
        scratch_types=[
            pltpu.VMEM((WPW + 1, 16), jnp.int32),
            pltpu.VMEM((D, WINW), jnp.float32),
            pltpu.VMEM((D, WINW), jnp.float32),
            pltpu.VMEM((D, D), jnp.float32),
            pltpu.VMEM((4 * CH,), jnp.int32),
            pltpu.VMEM((4 * CH,), jnp.int32),
            pltpu.VMEM((2 * CH,), jnp.int32),
            pltpu.VMEM((2 * CH,), jnp.int32),
            pltpu.VMEM((2 * CH,), jnp.int32),
            pltpu.VMEM((CH,), jnp.int32),
            pltpu.VMEM((2 * CH, W), jnp.float32),
            pltpu.VMEM((2 * CH, W), jnp.float32),
            pltpu.VMEM((CH, W), jnp.float32),
            pltpu.SemaphoreType.DMA,
            pltpu.SemaphoreType.DMA,
            pltpu.SemaphoreType.DMA,
            pltpu.SemaphoreType.DMA,
            pltpu.SemaphoreType.DMA,
            pltpu.SemaphoreType.DMA,
            pltpu.SemaphoreType.DMA,
        ],
    )(_p1_body)
    gath = p1(tab, tail_tab, meta, cd)

    p2 = functools.partial(
        pl.kernel,
        mesh=mesh,
        compiler_params=cparams,
        out_type=jax.ShapeDtypeStruct((nw, LANES), jnp.float32),
        scratch_types=[
            pltpu.VMEM((3, NSUB, SUB), jnp.int32),
            pltpu.VMEM((SUB, W), jnp.float32),
            pltpu.VMEM((SUB, W), jnp.float32),
            pltpu.VMEM((SUB, W), jnp.float32),
            pltpu.VMEM((SUB, W), jnp.float32),
            pltpu.VMEM((SUB, W), jnp.float32),
            pltpu.VMEM((SUB, W), jnp.float32),
            pltpu.VMEM((LANES,), jnp.float32),
            pltpu.SemaphoreType.DMA,
        ],
    )(_p2_body)
    partials = p2(gath, small, idx3)
    return 1.0 + jnp.sum(partials) / B
